# baseline (speedup 1.0000x reference)
import functools

import jax
import jax.numpy as jnp
from jax import lax
from jax.experimental import pallas as pl
from jax.experimental.pallas import tpu as pltpu
from jax.experimental.pallas import tpu_sc as plsc

_CH = 128
_NW = 32
_L = 16
_K = 3
_KS = 16
_DB = 2000


def _mesh():
    return plsc.VectorSubcoreMesh(core_axis_name="c", subcore_axis_name="s")


def _sc_params():
    return dict(
        mesh=_mesh(),
        compiler_params=pltpu.CompilerParams(
            needs_layout_passes=False, use_tc_tiling_on_sc=False),
    )


def _rsqrt16(x):
    i = lax.bitcast_convert_type(x, jnp.int32)
    i = jnp.int32(0x5F3759DF) - lax.shift_right_arithmetic(i, 1)
    y = lax.bitcast_convert_type(i, jnp.float32)
    for _ in range(3):
        y = y * (1.5 - 0.5 * x * y * y)
    return y


def _build_dis(degp_h, dis, tmp, n_nodes):
    pltpu.sync_copy(degp_h.at[0], dis)
    pltpu.sync_copy(degp_h.at[1], tmp)

    def body(g, carry):
        sl = pl.ds(g * _L, _L)
        dis[sl] = _rsqrt16(dis[sl] + tmp[sl])
        return carry

    lax.fori_loop(0, n_nodes // _L, body, 0)


def _build_dis_blk(degp_h, dis, dtmp, n_nodes):
    pltpu.sync_copy(degp_h.at[0], dis)
    for blk in range(n_nodes // _DB):
        pltpu.sync_copy(degp_h.at[1, pl.ds(blk * _DB, _DB)], dtmp)

        def body(g, carry, blk=blk):
            sl_d = pl.ds(blk * _DB + g * _L, _L)
            sl_t = pl.ds(g * _L, _L)
            dis[sl_d] = _rsqrt16(dis[sl_d] + dtmp[sl_t])
            return carry

        lax.fori_loop(0, _DB // _L, body, 0)


def _sc_degrees(row2, col2, ew2, zeros_n, n_nodes, nchunks):
    @functools.partial(
        pl.kernel,
        out_type=(jax.ShapeDtypeStruct((2, n_nodes), jnp.float32),
                  jax.ShapeDtypeStruct((2, n_nodes), jnp.float32)),
        scratch_types=(
            pltpu.VMEM((nchunks, _CH), jnp.int32),
            pltpu.VMEM((nchunks, _CH), jnp.int32),
            pltpu.VMEM((nchunks, _CH), jnp.float32),
            pltpu.VMEM_SHARED((n_nodes,), jnp.float32),
            pltpu.VMEM_SHARED((n_nodes,), jnp.float32),
            pltpu.SemaphoreType.DMA,
        ),
        **_sc_params(),
    )
    def deg_kernel(row_h, col_h, ew_h, z_h, deg_o, degw_o,
                   ridx, cidx, ewv, dacc, wacc, sem):
        c = lax.axis_index("c")
        s = lax.axis_index("s")
        wid = s * 2 + c

        @pl.when(s == 0)
        def _init():
            pltpu.sync_copy(z_h, dacc)
            pltpu.sync_copy(z_h, wacc)

        pltpu.sync_copy(row_h.at[pl.ds(wid * nchunks, nchunks)], ridx)
        pltpu.sync_copy(col_h.at[pl.ds(wid * nchunks, nchunks)], cidx)
        pltpu.sync_copy(ew_h.at[pl.ds(wid * nchunks, nchunks)], ewv)
        plsc.subcore_barrier()

        def batch(j0, nb):
            ds_ = []
            for b in range(nb):
                j = j0 + b
                ds_.append(pltpu.async_copy(
                    ewv.at[j], dacc.at[cidx.at[j]], sem, add=True))
                ds_.append(pltpu.async_copy(
                    ewv.at[j], wacc.at[ridx.at[j]], sem, add=True))
            for d in ds_:
                d.wait()

        def body(jj, carry):
            batch(jj * _KS, _KS)
            return carry

        lax.fori_loop(0, nchunks // _KS, body, 0)
        if nchunks % _KS:
            batch(nchunks - nchunks % _KS, nchunks % _KS)
        plsc.subcore_barrier()

        @pl.when(s == 0)
        def _out():
            pltpu.sync_copy(dacc, deg_o.at[c])
            pltpu.sync_copy(wacc, degw_o.at[c])

    return deg_kernel(row2, col2, ew2, zeros_n)


def _sc_msg64(row2, col2, ew2, xw, degp, zeros_nd, n_nodes, dh, nchunks):
    nsuper = nchunks // _K

    @functools.partial(
        pl.kernel,
        out_type=jax.ShapeDtypeStruct((2, n_nodes, dh), jnp.float32),
        scratch_types=(
            pltpu.VMEM((2, _K, _CH), jnp.int32),
            pltpu.VMEM((2, _K, _CH), jnp.int32),
            pltpu.VMEM((2, _K, _CH), jnp.float32),
            pltpu.VMEM((_K, _CH, dh), jnp.float32),
            pltpu.VMEM((n_nodes,), jnp.float32),
            pltpu.VMEM((_DB,), jnp.float32),
            pltpu.VMEM_SHARED((n_nodes, dh), jnp.float32),
            pltpu.VMEM_SHARED((n_nodes, dh), jnp.float32),
            pltpu.SemaphoreType.DMA,
            pltpu.SemaphoreType.DMA,
            pltpu.SemaphoreType.DMA,
        ),
        **_sc_params(),
    )
    def msg_kernel(row_h, col_h, ew_h, xw_h, degp_h, z2_h, acc_o,
                   ridx, cidx, ewv, rows, dis, tmp, xw_s, acc,
                   gsem, ssem, isem):
        c = lax.axis_index("c")
        s = lax.axis_index("s")
        wid = s * 2 + c
        cb = wid * nchunks

        @pl.when(s == 0)
        def _init():
            pltpu.sync_copy(z2_h, acc)

        @pl.when(s == 1)
        def _stage():
            pltpu.sync_copy(xw_h, xw_s)

        _build_dis_blk(degp_h, dis, tmp, n_nodes)
        pltpu.sync_copy(row_h.at[pl.ds(cb, _K)], ridx.at[0])
        pltpu.sync_copy(col_h.at[pl.ds(cb, _K)], cidx.at[0])
        pltpu.sync_copy(ew_h.at[pl.ds(cb, _K)], ewv.at[0])
        plsc.subcore_barrier()

        def process(p):
            gds = [pltpu.async_copy(xw_s.at[ridx.at[p, b]], rows.at[b], gsem)
                   for b in range(_K)]
            for d in gds:
                d.wait()
            for b in range(_K):
                rb = rows.at[b]

                def sbody(g, cc, p=p, b=b, rb=rb):
                    sl = pl.ds(g * _L, _L)
                    e0 = g * _L
                    a16 = (ewv[p, b, sl]
                           * plsc.load_gather(dis, [ridx[p, b, sl]])
                           * plsc.load_gather(dis, [cidx[p, b, sl]]))
                    for l in range(_L):
                        aa = a16[l]
                        for q in range(dh // _L):
                            fs = pl.ds(q * _L, _L)
                            rb[e0 + l, fs] = rb[e0 + l, fs] * aa
                    return cc

                lax.fori_loop(0, _CH // _L, sbody, 0)
            sds = [pltpu.async_copy(
                       rows.at[b], acc.at[cidx.at[p, b]], ssem, add=True)
                   for b in range(_K)]
            for d in sds:
                d.wait()

        def prefetch(jj, p):
            j = jnp.minimum(jj, nsuper - 1) * _K + cb
            return [pltpu.async_copy(row_h.at[pl.ds(j, _K)], ridx.at[p], isem),
                    pltpu.async_copy(col_h.at[pl.ds(j, _K)], cidx.at[p], isem),
                    pltpu.async_copy(ew_h.at[pl.ds(j, _K)], ewv.at[p], isem)]

        def pair_body(t, carry):
            jj0 = 2 * t
            i1 = prefetch(jj0 + 1, 1)
            process(0)
            for d in i1:
                d.wait()
            i0 = prefetch(jj0 + 2, 0)
            process(1)
            for d in i0:
                d.wait()
            return carry

        lax.fori_loop(0, nsuper // 2, pair_body, 0)
        if nsuper % 2:
            process(0)
        plsc.subcore_barrier()

        @pl.when(s == 0)
        def _out():
            pltpu.sync_copy(acc, acc_o.at[c])

    return msg_kernel(row2, col2, ew2, xw, degp, zeros_nd)


def _sc_msg1(row2, col2, ew2, z, degp, zeros_n, n_nodes, nchunks):
    @functools.partial(
        pl.kernel,
        out_type=jax.ShapeDtypeStruct((2, n_nodes), jnp.float32),
        scratch_types=(
            pltpu.VMEM((nchunks, _CH), jnp.int32),
            pltpu.VMEM((nchunks, _CH), jnp.int32),
            pltpu.VMEM((nchunks, _CH), jnp.float32),
            pltpu.VMEM((_KS, _CH), jnp.float32),
            pltpu.VMEM((n_nodes,), jnp.float32),
            pltpu.VMEM((n_nodes,), jnp.float32),
            pltpu.VMEM((n_nodes,), jnp.float32),
            pltpu.VMEM_SHARED((n_nodes,), jnp.float32),
            pltpu.SemaphoreType.DMA,
        ),
        **_sc_params(),
    )
    def msg1_kernel(row_h, col_h, ew_h, z_h, degp_h, zz_h, acc_o,
                    ridx, cidx, ewv, val, ztab, dis, tmp, acc, ssem):
        c = lax.axis_index("c")
        s = lax.axis_index("s")
        wid = s * 2 + c

        @pl.when(s == 0)
        def _init():
            pltpu.sync_copy(zz_h, acc)

        pltpu.sync_copy(row_h.at[pl.ds(wid * nchunks, nchunks)], ridx)
        pltpu.sync_copy(col_h.at[pl.ds(wid * nchunks, nchunks)], cidx)
        pltpu.sync_copy(ew_h.at[pl.ds(wid * nchunks, nchunks)], ewv)
        _build_dis(degp_h, dis, tmp, n_nodes)
        pltpu.sync_copy(z_h, ztab)
        plsc.subcore_barrier()

        def batch(j0, nb):
            for b in range(nb):
                j = j0 + b

                def gbody(g, cc, j=j, b=b):
                    sl = pl.ds(g * _L, _L)
                    r16 = ridx[j, sl]
                    a16 = (ewv[j, sl] * plsc.load_gather(dis, [r16])
                           * plsc.load_gather(dis, [cidx[j, sl]]))
                    val[b, sl] = a16 * plsc.load_gather(ztab, [r16])
                    return cc

                lax.fori_loop(0, _CH // _L, gbody, 0)
            sds = [pltpu.async_copy(
                       val.at[b], acc.at[cidx.at[j0 + b]], ssem, add=True)
                   for b in range(nb)]
            for d in sds:
                d.wait()

        def body(jj, carry):
            batch(jj * _KS, _KS)
            return carry

        lax.fori_loop(0, nchunks // _KS, body, 0)
        if nchunks % _KS:
            batch(nchunks - nchunks % _KS, nchunks % _KS)
        plsc.subcore_barrier()

        @pl.when(s == 0)
        def _out():
            pltpu.sync_copy(acc, acc_o.at[c])

    return msg1_kernel(row2, col2, ew2, z, degp, zeros_n)


def _tc_matmul(x, W1):
    def body(x_ref, w_ref, o_ref):
        o_ref[...] = jnp.dot(x_ref[...], w_ref[...],
                             preferred_element_type=jnp.float32)

    return pl.pallas_call(
        body,
        out_shape=jax.ShapeDtypeStruct((x.shape[0], W1.shape[1]), jnp.float32),
    )(x, W1)


def _tc_layer2(accp, b1, W2):
    n, dh = accp.shape[1], accp.shape[2]

    def body(a_ref, b1_ref, w2_ref, o_ref):
        h = jnp.maximum(a_ref[0] + a_ref[1] + b1_ref[...], 0.0)
        o_ref[...] = jnp.dot(h, w2_ref[...], preferred_element_type=jnp.float32)

    return pl.pallas_call(
        body,
        out_shape=jax.ShapeDtypeStruct((n, 1), jnp.float32),
    )(accp, b1.reshape(1, dh), W2)


def _tc_final(acc2p, degwp, b2):
    n = acc2p.shape[2]

    def body(a_ref, d_ref, b_ref, o_ref):
        sarg = a_ref[0] + a_ref[1] + b_ref[...]
        score = 1.0 / (1.0 + jnp.exp(-sarg))
        degw = d_ref[0] + d_ref[1] - 1.0
        o_ref[...] = score * (1.0 + degw / jnp.max(degw))

    return pl.pallas_call(
        body,
        out_shape=jax.ShapeDtypeStruct((1, n), jnp.float32),
    )(acc2p, degwp, b2)


def kernel(x, edge_index, edge_weight, W1, b1, W2, b2):
    n = x.shape[0]
    dh = W1.shape[1]
    e = edge_weight.shape[0]
    row = edge_index[0]
    col = edge_index[1]

    loop = jnp.arange(n, dtype=row.dtype)
    e_tot = e + n
    unit = _NW * _K * _CH
    epw = (-(-e_tot // unit) * unit) // _NW
    pad = _NW * epw - e_tot
    zi = jnp.zeros((pad,), row.dtype)
    row2 = jnp.concatenate([row, loop, zi]).reshape(-1, _CH)
    col2 = jnp.concatenate([col, loop, zi]).reshape(-1, _CH)
    ew2 = jnp.concatenate([edge_weight, jnp.ones((n,), x.dtype),
                           jnp.zeros((pad,), x.dtype)]).reshape(-1, _CH)
    zeros_n = jnp.zeros((n,), jnp.float32)
    zeros_nd = jnp.zeros((n, dh), jnp.float32)
    nchunks = epw // _CH

    degp, degwp = _sc_degrees(row2, col2, ew2, zeros_n, n, nchunks)
    xw = _tc_matmul(x, W1)
    accp = _sc_msg64(row2, col2, ew2, xw, degp, zeros_nd, n, dh, nchunks)
    z = _tc_layer2(accp, b1, W2).reshape(n)
    acc2p = _sc_msg1(row2, col2, ew2, z, degp, zeros_n, n, nchunks)
    out = _tc_final(acc2p.reshape(2, 1, n), degwp.reshape(2, 1, n),
                    b2.reshape(1, 1))
    return out.reshape(n)


# speedup vs baseline: 1.0088x; 1.0088x over previous
import functools

import jax
import jax.numpy as jnp
from jax import lax
from jax.experimental import pallas as pl
from jax.experimental.pallas import tpu as pltpu
from jax.experimental.pallas import tpu_sc as plsc

_CH = 128
_NW = 32
_L = 16
_K = 3
_KS = 8
_DB = 2000


def _mesh():
    return plsc.VectorSubcoreMesh(core_axis_name="c", subcore_axis_name="s")


def _sc_params():
    return dict(
        mesh=_mesh(),
        compiler_params=pltpu.CompilerParams(
            needs_layout_passes=False, use_tc_tiling_on_sc=False),
    )


def _rsqrt16(x):
    i = lax.bitcast_convert_type(x, jnp.int32)
    i = jnp.int32(0x5F3759DF) - lax.shift_right_arithmetic(i, 1)
    y = lax.bitcast_convert_type(i, jnp.float32)
    for _ in range(3):
        y = y * (1.5 - 0.5 * x * y * y)
    return y


def _build_dis(degp_h, dis, tmp, n_nodes):
    pltpu.sync_copy(degp_h.at[0], dis)
    pltpu.sync_copy(degp_h.at[1], tmp)

    def body(g, carry):
        sl = pl.ds(g * _L, _L)
        dis[sl] = _rsqrt16(dis[sl] + tmp[sl])
        return carry

    lax.fori_loop(0, n_nodes // _L, body, 0)


def _build_dis_blk(degp_h, dis, dtmp, n_nodes):
    pltpu.sync_copy(degp_h.at[0], dis)
    for blk in range(n_nodes // _DB):
        pltpu.sync_copy(degp_h.at[1, pl.ds(blk * _DB, _DB)], dtmp)

        def body(g, carry, blk=blk):
            sl_d = pl.ds(blk * _DB + g * _L, _L)
            sl_t = pl.ds(g * _L, _L)
            dis[sl_d] = _rsqrt16(dis[sl_d] + dtmp[sl_t])
            return carry

        lax.fori_loop(0, _DB // _L, body, 0)


def _sc_degrees(row2, col2, ew2, zeros_n, n_nodes, nchunks):
    @functools.partial(
        pl.kernel,
        out_type=(jax.ShapeDtypeStruct((2, n_nodes), jnp.float32),
                  jax.ShapeDtypeStruct((2, n_nodes), jnp.float32)),
        scratch_types=(
            pltpu.VMEM((nchunks, _CH), jnp.int32),
            pltpu.VMEM((nchunks, _CH), jnp.int32),
            pltpu.VMEM((nchunks, _CH), jnp.float32),
            pltpu.VMEM_SHARED((n_nodes,), jnp.float32),
            pltpu.VMEM_SHARED((n_nodes,), jnp.float32),
            pltpu.SemaphoreType.DMA,
        ),
        **_sc_params(),
    )
    def deg_kernel(row_h, col_h, ew_h, z_h, deg_o, degw_o,
                   ridx, cidx, ewv, dacc, wacc, sem):
        c = lax.axis_index("c")
        s = lax.axis_index("s")
        wid = s * 2 + c

        @pl.when(s == 0)
        def _init():
            pltpu.sync_copy(z_h, dacc)
            pltpu.sync_copy(z_h, wacc)

        pltpu.sync_copy(row_h.at[pl.ds(wid * nchunks, nchunks)], ridx)
        pltpu.sync_copy(col_h.at[pl.ds(wid * nchunks, nchunks)], cidx)
        pltpu.sync_copy(ew_h.at[pl.ds(wid * nchunks, nchunks)], ewv)
        plsc.subcore_barrier()

        def batch(j0, nb):
            ds_ = []
            for b in range(nb):
                j = j0 + b
                ds_.append(pltpu.async_copy(
                    ewv.at[j], dacc.at[cidx.at[j]], sem, add=True))
                ds_.append(pltpu.async_copy(
                    ewv.at[j], wacc.at[ridx.at[j]], sem, add=True))
            for d in ds_:
                d.wait()

        def body(jj, carry):
            batch(jj * _KS, _KS)
            return carry

        lax.fori_loop(0, nchunks // _KS, body, 0)
        if nchunks % _KS:
            batch(nchunks - nchunks % _KS, nchunks % _KS)
        plsc.subcore_barrier()

        @pl.when(s == 0)
        def _out():
            pltpu.sync_copy(dacc, deg_o.at[c])
            pltpu.sync_copy(wacc, degw_o.at[c])

    return deg_kernel(row2, col2, ew2, zeros_n)


def _sc_msg64(row2, col2, ew2, xw, degp, zeros_nd, n_nodes, dh, nchunks):
    nsuper = nchunks // _K

    @functools.partial(
        pl.kernel,
        out_type=jax.ShapeDtypeStruct((2, n_nodes, dh), jnp.float32),
        scratch_types=(
            pltpu.VMEM((2, _K, _CH), jnp.int32),
            pltpu.VMEM((2, _K, _CH), jnp.int32),
            pltpu.VMEM((2, _K, _CH), jnp.float32),
            pltpu.VMEM((_K, _CH, dh), jnp.float32),
            pltpu.VMEM((n_nodes,), jnp.float32),
            pltpu.VMEM((_DB,), jnp.float32),
            pltpu.VMEM_SHARED((n_nodes, dh), jnp.float32),
            pltpu.VMEM_SHARED((n_nodes, dh), jnp.float32),
            pltpu.SemaphoreType.DMA,
            pltpu.SemaphoreType.DMA,
            pltpu.SemaphoreType.DMA,
        ),
        **_sc_params(),
    )
    def msg_kernel(row_h, col_h, ew_h, xw_h, degp_h, z2_h, acc_o,
                   ridx, cidx, ewv, rows, dis, tmp, xw_s, acc,
                   gsem, ssem, isem):
        c = lax.axis_index("c")
        s = lax.axis_index("s")
        wid = s * 2 + c
        cb = wid * nchunks

        @pl.when(s == 0)
        def _init():
            pltpu.sync_copy(z2_h, acc)

        @pl.when(s == 1)
        def _stage():
            pltpu.sync_copy(xw_h, xw_s)

        _build_dis_blk(degp_h, dis, tmp, n_nodes)
        pltpu.sync_copy(row_h.at[pl.ds(cb, _K)], ridx.at[0])
        pltpu.sync_copy(col_h.at[pl.ds(cb, _K)], cidx.at[0])
        pltpu.sync_copy(ew_h.at[pl.ds(cb, _K)], ewv.at[0])
        plsc.subcore_barrier()

        def process(p):
            gds = [pltpu.async_copy(xw_s.at[ridx.at[p, b]], rows.at[b], gsem)
                   for b in range(_K)]
            for d in gds:
                d.wait()
            for b in range(_K):
                rb = rows.at[b]

                def sbody(g, cc, p=p, b=b, rb=rb):
                    sl = pl.ds(g * _L, _L)
                    e0 = g * _L
                    a16 = (ewv[p, b, sl]
                           * plsc.load_gather(dis, [ridx[p, b, sl]])
                           * plsc.load_gather(dis, [cidx[p, b, sl]]))
                    for l in range(_L):
                        aa = a16[l]
                        for q in range(dh // _L):
                            fs = pl.ds(q * _L, _L)
                            rb[e0 + l, fs] = rb[e0 + l, fs] * aa
                    return cc

                lax.fori_loop(0, _CH // _L, sbody, 0)
            sds = [pltpu.async_copy(
                       rows.at[b], acc.at[cidx.at[p, b]], ssem, add=True)
                   for b in range(_K)]
            for d in sds:
                d.wait()

        def prefetch(jj, p):
            j = jnp.minimum(jj, nsuper - 1) * _K + cb
            return [pltpu.async_copy(row_h.at[pl.ds(j, _K)], ridx.at[p], isem),
                    pltpu.async_copy(col_h.at[pl.ds(j, _K)], cidx.at[p], isem),
                    pltpu.async_copy(ew_h.at[pl.ds(j, _K)], ewv.at[p], isem)]

        def pair_body(t, carry):
            jj0 = 2 * t
            i1 = prefetch(jj0 + 1, 1)
            process(0)
            for d in i1:
                d.wait()
            i0 = prefetch(jj0 + 2, 0)
            process(1)
            for d in i0:
                d.wait()
            return carry

        lax.fori_loop(0, nsuper // 2, pair_body, 0)
        if nsuper % 2:
            process(0)
        plsc.subcore_barrier()

        @pl.when(s == 0)
        def _out():
            pltpu.sync_copy(acc, acc_o.at[c])

    return msg_kernel(row2, col2, ew2, xw, degp, zeros_nd)


def _sc_msg1(row2, col2, ew2, z, degp, zeros_n, n_nodes, nchunks):
    @functools.partial(
        pl.kernel,
        out_type=jax.ShapeDtypeStruct((2, n_nodes), jnp.float32),
        scratch_types=(
            pltpu.VMEM((nchunks, _CH), jnp.int32),
            pltpu.VMEM((nchunks, _CH), jnp.int32),
            pltpu.VMEM((nchunks, _CH), jnp.float32),
            pltpu.VMEM((_KS, _CH), jnp.float32),
            pltpu.VMEM((n_nodes,), jnp.float32),
            pltpu.VMEM((n_nodes,), jnp.float32),
            pltpu.VMEM((n_nodes,), jnp.float32),
            pltpu.VMEM_SHARED((n_nodes,), jnp.float32),
            pltpu.SemaphoreType.DMA,
        ),
        **_sc_params(),
    )
    def msg1_kernel(row_h, col_h, ew_h, z_h, degp_h, zz_h, acc_o,
                    ridx, cidx, ewv, val, ztab, dis, tmp, acc, ssem):
        c = lax.axis_index("c")
        s = lax.axis_index("s")
        wid = s * 2 + c

        @pl.when(s == 0)
        def _init():
            pltpu.sync_copy(zz_h, acc)

        pltpu.sync_copy(row_h.at[pl.ds(wid * nchunks, nchunks)], ridx)
        pltpu.sync_copy(col_h.at[pl.ds(wid * nchunks, nchunks)], cidx)
        pltpu.sync_copy(ew_h.at[pl.ds(wid * nchunks, nchunks)], ewv)
        _build_dis(degp_h, dis, tmp, n_nodes)
        pltpu.sync_copy(z_h, ztab)
        plsc.subcore_barrier()

        def batch(j0, nb):
            for b in range(nb):
                j = j0 + b

                def gbody(g, cc, j=j, b=b):
                    sl = pl.ds(g * _L, _L)
                    r16 = ridx[j, sl]
                    a16 = (ewv[j, sl] * plsc.load_gather(dis, [r16])
                           * plsc.load_gather(dis, [cidx[j, sl]]))
                    val[b, sl] = a16 * plsc.load_gather(ztab, [r16])
                    return cc

                lax.fori_loop(0, _CH // _L, gbody, 0)
            sds = [pltpu.async_copy(
                       val.at[b], acc.at[cidx.at[j0 + b]], ssem, add=True)
                   for b in range(nb)]
            for d in sds:
                d.wait()

        def body(jj, carry):
            batch(jj * _KS, _KS)
            return carry

        lax.fori_loop(0, nchunks // _KS, body, 0)
        if nchunks % _KS:
            batch(nchunks - nchunks % _KS, nchunks % _KS)
        plsc.subcore_barrier()

        @pl.when(s == 0)
        def _out():
            pltpu.sync_copy(acc, acc_o.at[c])

    return msg1_kernel(row2, col2, ew2, z, degp, zeros_n)


def _tc_matmul(x, W1):
    def body(x_ref, w_ref, o_ref):
        o_ref[...] = jnp.dot(x_ref[...], w_ref[...],
                             preferred_element_type=jnp.float32)

    return pl.pallas_call(
        body,
        out_shape=jax.ShapeDtypeStruct((x.shape[0], W1.shape[1]), jnp.float32),
    )(x, W1)


def _tc_layer2(accp, b1, W2):
    n, dh = accp.shape[1], accp.shape[2]

    def body(a_ref, b1_ref, w2_ref, o_ref):
        h = jnp.maximum(a_ref[0] + a_ref[1] + b1_ref[...], 0.0)
        o_ref[...] = jnp.dot(h, w2_ref[...], preferred_element_type=jnp.float32)

    return pl.pallas_call(
        body,
        out_shape=jax.ShapeDtypeStruct((n, 1), jnp.float32),
    )(accp, b1.reshape(1, dh), W2)


def _tc_final(acc2p, degwp, b2):
    n = acc2p.shape[2]

    def body(a_ref, d_ref, b_ref, o_ref):
        sarg = a_ref[0] + a_ref[1] + b_ref[...]
        score = 1.0 / (1.0 + jnp.exp(-sarg))
        degw = d_ref[0] + d_ref[1] - 1.0
        o_ref[...] = score * (1.0 + degw / jnp.max(degw))

    return pl.pallas_call(
        body,
        out_shape=jax.ShapeDtypeStruct((1, n), jnp.float32),
    )(acc2p, degwp, b2)


def kernel(x, edge_index, edge_weight, W1, b1, W2, b2):
    n = x.shape[0]
    dh = W1.shape[1]
    e = edge_weight.shape[0]
    row = edge_index[0]
    col = edge_index[1]

    loop = jnp.arange(n, dtype=row.dtype)
    e_tot = e + n
    unit = _NW * _K * _CH
    epw = (-(-e_tot // unit) * unit) // _NW
    pad = _NW * epw - e_tot
    zi = jnp.zeros((pad,), row.dtype)
    row2 = jnp.concatenate([row, loop, zi]).reshape(-1, _CH)
    col2 = jnp.concatenate([col, loop, zi]).reshape(-1, _CH)
    ew2 = jnp.concatenate([edge_weight, jnp.ones((n,), x.dtype),
                           jnp.zeros((pad,), x.dtype)]).reshape(-1, _CH)
    zeros_n = jnp.zeros((n,), jnp.float32)
    zeros_nd = jnp.zeros((n, dh), jnp.float32)
    nchunks = epw // _CH

    degp, degwp = _sc_degrees(row2, col2, ew2, zeros_n, n, nchunks)
    xw = _tc_matmul(x, W1)
    accp = _sc_msg64(row2, col2, ew2, xw, degp, zeros_nd, n, dh, nchunks)
    z = _tc_layer2(accp, b1, W2).reshape(n)
    acc2p = _sc_msg1(row2, col2, ew2, z, degp, zeros_n, n, nchunks)
    out = _tc_final(acc2p.reshape(2, 1, n), degwp.reshape(2, 1, n),
                    b2.reshape(1, 1))
    return out.reshape(n)
